# Initial kernel scaffold; baseline (speedup 1.0000x reference)
import jax
import jax.numpy as jnp
from jax.experimental import pallas as pl


def kernel(boxes, scores):
    raise NotImplementedError("write your pallas kernel here")


# speedup vs baseline: 15.9062x; 15.9062x over previous
import functools

import jax
import jax.numpy as jnp
from jax.experimental import pallas as pl
from jax.experimental.pallas import tpu as pltpu

N = 20000
DET = 100
SCORE_THRESH = 0.05
NMS_THRESH = 0.5
NEG = -1e9

ROWS = 160
LANES = 128


def _nms_body(x1_ref, y1_ref, x2_ref, y2_ref, s_ref, out_ref, sc_ref):
    sc_ref[...] = jnp.where(s_ref[...] > SCORE_THRESH, s_ref[...], NEG)

    flat_iota = (
        jax.lax.broadcasted_iota(jnp.int32, (ROWS, LANES), 0) * LANES
        + jax.lax.broadcasted_iota(jnp.int32, (ROWS, LANES), 1)
    )
    lane = jax.lax.broadcasted_iota(jnp.int32, (1, LANES), 1)

    def step(i, _):
        sc = sc_ref[...]
        m = jnp.max(sc)
        idx = jnp.min(jnp.where(sc == m, flat_iota, jnp.int32(2**31 - 1)))
        eq = flat_iota == idx
        eqf = eq.astype(jnp.float32)
        x1 = x1_ref[...]
        y1 = y1_ref[...]
        x2 = x2_ref[...]
        y2 = y2_ref[...]
        bx1 = jnp.sum(x1 * eqf)
        by1 = jnp.sum(y1 * eqf)
        bx2 = jnp.sum(x2 * eqf)
        by2 = jnp.sum(y2 * eqf)

        xx1 = jnp.maximum(bx1, x1)
        yy1 = jnp.maximum(by1, y1)
        xx2 = jnp.minimum(bx2, x2)
        yy2 = jnp.minimum(by2, y2)
        inter = jnp.maximum(xx2 - xx1, 0.0) * jnp.maximum(yy2 - yy1, 0.0)
        barea = (bx2 - bx1) * (by2 - by1)
        area = (x2 - x1) * (y2 - y1)
        iou = inter / (barea + area - inter + 1e-9)
        suppress = (iou > NMS_THRESH) | eq
        sc_ref[...] = jnp.where(suppress, NEG, sc)

        valid = (m > SCORE_THRESH).astype(jnp.float32)
        row = jnp.where(lane == 0, bx1,
              jnp.where(lane == 1, by1,
              jnp.where(lane == 2, bx2,
              jnp.where(lane == 3, by2, m)))) * valid
        out_ref[pl.ds(i, 1), :] = row
        return 0

    jax.lax.fori_loop(0, DET, step, 0)


@jax.jit
def kernel(boxes, scores):
    pad = ROWS * LANES - N
    x1 = jnp.pad(boxes[:, 0], (0, pad)).reshape(ROWS, LANES)
    y1 = jnp.pad(boxes[:, 1], (0, pad)).reshape(ROWS, LANES)
    x2 = jnp.pad(boxes[:, 2], (0, pad)).reshape(ROWS, LANES)
    y2 = jnp.pad(boxes[:, 3], (0, pad)).reshape(ROWS, LANES)
    s = jnp.pad(scores, (0, pad)).reshape(ROWS, LANES)

    out = pl.pallas_call(
        _nms_body,
        out_shape=jax.ShapeDtypeStruct((DET, LANES), jnp.float32),
        scratch_shapes=[pltpu.VMEM((ROWS, LANES), jnp.float32)],
    )(x1, y1, x2, y2, s)
    return out[:, :5]
